# Initial kernel scaffold; baseline (speedup 1.0000x reference)
import jax
import jax.numpy as jnp
from jax.experimental import pallas as pl


def kernel(x, weight, bias):
    raise NotImplementedError("write your pallas kernel here")


# speedup vs baseline: 1.4042x; 1.4042x over previous
import jax
import jax.numpy as jnp
from jax.experimental import pallas as pl
from jax.experimental.pallas import tpu as pltpu


def _interleave_matmul_kernel(x_ref, w_ref, b_ref, o_ref):
    y = jnp.dot(x_ref[...], w_ref[...], preferred_element_type=jnp.float32)
    o_ref[...] = (y + b_ref[0, 0]).astype(o_ref.dtype)


def kernel(x, weight, bias):
    B, D = x.shape
    dtype = x.dtype
    L = 128
    Bp = ((B + L - 1) // L) * L
    if Bp != B:
        x = jnp.pad(x, ((0, Bp - B), (0, 0)))
    R = Bp // L
    K = L * D

    xf = x.reshape(R, K)

    j = jnp.arange(K, dtype=jnp.int32)
    w_rep = jnp.tile(weight.reshape(-1), L)
    sel = (j // D)[:, None] == jnp.arange(L, dtype=jnp.int32)[None, :]
    w2 = jnp.where(sel, w_rep[:, None], jnp.zeros((), weight.dtype)).astype(dtype)
    b2 = bias.reshape(1, 1).astype(jnp.float32)

    TR = min(256, R)
    out = pl.pallas_call(
        _interleave_matmul_kernel,
        out_shape=jax.ShapeDtypeStruct((R, L), dtype),
        grid=(pl.cdiv(R, TR),),
        in_specs=[
            pl.BlockSpec((TR, K), lambda i: (i, 0)),
            pl.BlockSpec((K, L), lambda i: (0, 0)),
            pl.BlockSpec(memory_space=pltpu.SMEM),
        ],
        out_specs=pl.BlockSpec((TR, L), lambda i: (i, 0)),
        compiler_params=pltpu.CompilerParams(
            dimension_semantics=("parallel",)),
    )(xf, w2, b2)

    return out.reshape(Bp, 1)[:B]


# speedup vs baseline: 1.6067x; 1.1442x over previous
import jax
import jax.numpy as jnp
from jax.experimental import pallas as pl
from jax.experimental.pallas import tpu as pltpu


def _rowvec_matmul_kernel(x_ref, w_ref, b_ref, o_ref):
    yrow = jax.lax.dot_general(
        w_ref[...], x_ref[...],
        dimension_numbers=(((1,), (1,)), ((), ())),
        preferred_element_type=jnp.float32)
    y = yrow + b_ref[0, 0]
    o_ref[...] = y.reshape(o_ref.shape).astype(o_ref.dtype)


def kernel(x, weight, bias):
    B, D = x.shape
    dtype = x.dtype
    L = 128
    Bp = ((B + L - 1) // L) * L
    if Bp != B:
        x = jnp.pad(x, ((0, Bp - B), (0, 0)))

    w2d = weight.reshape(1, D).astype(dtype)
    b2 = bias.reshape(1, 1).astype(jnp.float32)

    TB = min(8192, Bp)
    out = pl.pallas_call(
        _rowvec_matmul_kernel,
        out_shape=jax.ShapeDtypeStruct((Bp // L, L), dtype),
        grid=(Bp // TB,),
        in_specs=[
            pl.BlockSpec((TB, D), lambda i: (i, 0)),
            pl.BlockSpec((1, D), lambda i: (0, 0)),
            pl.BlockSpec(memory_space=pltpu.SMEM),
        ],
        out_specs=pl.BlockSpec((TB // L, L), lambda i: (i, 0)),
        compiler_params=pltpu.CompilerParams(
            dimension_semantics=("parallel",)),
    )(x, w2d, b2)

    return out.reshape(Bp, 1)[:B]
